# baseline (speedup 1.0000x reference)
import jax
import jax.numpy as jnp
from jax import lax
from jax.experimental import pallas as pl
from jax.experimental.pallas import tpu as pltpu
from jax.experimental.pallas import tpu_sc as plsc

F32 = jnp.float32
CHUNK = 80
ROWBLK = 80


def _atom_enc_body(x_ref, w_ref, b_ref, frag_ref, femb_ref, o_ref, xf_ref):
    o_ref[...] = (
        jnp.dot(x_ref[...], w_ref[...], preferred_element_type=F32, precision=lax.Precision.HIGHEST) + b_ref[...]
    )
    nv = femb_ref.shape[0]
    oh = (frag_ref[...] == lax.broadcasted_iota(jnp.int32, (1, nv), 1)).astype(F32)
    xf_ref[...] = jnp.dot(oh, femb_ref[...], preferred_element_type=F32, precision=lax.Precision.HIGHEST)


def _edge_enc_body(ea_ref, w1_ref, b1_ref, w2_ref, b2_ref, e1_ref, e2_ref):
    ea = ea_ref[...]
    e1_ref[...] = jnp.dot(ea, w1_ref[...], preferred_element_type=F32, precision=lax.Precision.HIGHEST) + b1_ref[...]
    e2_ref[...] = jnp.dot(ea, w2_ref[...], preferred_element_type=F32, precision=lax.Precision.HIGHEST) + b2_ref[...]


def _fold_body(h0_ref, s_ref, c_ref, o_ref):
    s = s_ref[0] + s_ref[1]
    c = c_ref[0] + c_ref[1]
    o_ref[...] = h0_ref[...] + s / jnp.maximum(c, 1.0)


def _layer_body(h_ref, p_ref, eps_ref, w1_ref, b1_ref, g1_ref, be1_ref,
                w2_ref, b2_ref, g2_ref, be2_ref, o_ref):
    z = h_ref[...] * (1.0 + eps_ref[0, 0]) + p_ref[0] + p_ref[1]
    a = jnp.dot(z, w1_ref[...], preferred_element_type=F32, precision=lax.Precision.HIGHEST) + b1_ref[...]
    m = jnp.mean(a, axis=0, keepdims=True)
    v = jnp.mean((a - m) ** 2, axis=0, keepdims=True)
    a = (a - m) * lax.rsqrt(v + 1e-5) * g1_ref[...] + be1_ref[...]
    a = jnp.maximum(a, 0.0)
    b = jnp.dot(a, w2_ref[...], preferred_element_type=F32, precision=lax.Precision.HIGHEST) + b2_ref[...]
    m2 = jnp.mean(b, axis=0, keepdims=True)
    v2 = jnp.mean((b - m2) ** 2, axis=0, keepdims=True)
    b = (b - m2) * lax.rsqrt(v2 + 1e-5) * g2_ref[...] + be2_ref[...]
    o_ref[...] = jnp.maximum(b, 0.0)


def _head_body(h_ref, batch_ref, aw1_ref, ab1_ref, aw2_ref, ab2_ref,
               ow1_ref, ob1_ref, ow2_ref, ob2_ref, o_ref):
    h = jnp.maximum(
        jnp.dot(h_ref[...], aw1_ref[...], preferred_element_type=F32, precision=lax.Precision.HIGHEST) + ab1_ref[...],
        0.0)
    h = jnp.maximum(
        jnp.dot(h, aw2_ref[...], preferred_element_type=F32, precision=lax.Precision.HIGHEST) + ab2_ref[...], 0.0)
    nb = o_ref.shape[0]
    seg = lax.broadcasted_iota(jnp.int32, (nb, 1), 0)
    oht = (seg == batch_ref[...]).astype(F32)
    gs = jnp.dot(oht, h, preferred_element_type=F32, precision=lax.Precision.HIGHEST)
    gc = jnp.sum(oht, axis=1, keepdims=True)
    g = gs / jnp.maximum(gc, 1.0)
    g = jnp.maximum(
        jnp.dot(g, ow1_ref[...], preferred_element_type=F32, precision=lax.Precision.HIGHEST) + ob1_ref[...], 0.0)
    o_ref[...] = jnp.dot(g, ow2_ref[...], preferred_element_type=F32, precision=lax.Precision.HIGHEST) + ob2_ref[...]


def _fill_zeros(zb_ref):
    ncol = zb_ref.shape[1]

    def body(r, _):
        for c in range(ncol // 16):
            zb_ref[r, pl.ds(c * 16, 16)] = jnp.zeros((16,), F32)
        return 0

    lax.fori_loop(0, zb_ref.shape[0], body, 0)


def _zero_table(zb_ref, tab_sh, sid, nrb):
    for j in range(-(-nrb // 16)):
        b = j * 16 + sid

        def _zb(b=b):
            pltpu.sync_copy(zb_ref, tab_sh.at[pl.ds(b * ROWBLK, ROWBLK)])

        pl.when(b < nrb)(_zb)


def _writeout_table(zb_ref, tab_sh, out_hbm, cid, sid, nrb):
    for j in range(-(-nrb // 16)):
        b = j * 16 + sid

        def _wb(b=b):
            base = b * ROWBLK
            pltpu.sync_copy(tab_sh.at[pl.ds(base, ROWBLK)], zb_ref)
            pltpu.sync_copy(zb_ref, out_hbm.at[cid, pl.ds(base, ROWBLK)])

        pl.when(b < nrb)(_wb)


def _segsum_sc_body(val_hbm, gidx_hbm, sidx_hbm, out_hbm,
                    inv, outv, rows, zb, tab_sh, sem):
    cid = lax.axis_index("c")
    sid = lax.axis_index("s")
    n = out_hbm.shape[1]
    nrb = n // ROWBLK
    nchunks = gidx_hbm.shape[0] // CHUNK
    wid = sid * 2 + cid

    _fill_zeros(zb)
    _zero_table(zb, tab_sh, sid, nrb)
    plsc.subcore_barrier()

    for j in range(-(-nchunks // 32)):
        chunk = j * 32 + wid

        def _do(chunk=chunk):
            off = chunk * CHUNK
            pltpu.sync_copy(gidx_hbm.at[pl.ds(off, CHUNK)], inv)
            pltpu.async_copy(val_hbm.at[inv], rows, sem).wait()
            pltpu.sync_copy(sidx_hbm.at[pl.ds(off, CHUNK)], outv.at[0])
            pltpu.sync_copy(rows, tab_sh.at[outv.at[0]], add=True)

        pl.when(chunk < nchunks)(_do)

    plsc.subcore_barrier()
    _writeout_table(zb, tab_sh, out_hbm, cid, sid, nrb)


def _count_sc_body(sidx_hbm, out_hbm, outv, ones_b, zb, tab_sh, sem):
    cid = lax.axis_index("c")
    sid = lax.axis_index("s")
    n = out_hbm.shape[1]
    nrb = n // ROWBLK
    nchunks = sidx_hbm.shape[0] // CHUNK
    wid = sid * 2 + cid

    _fill_zeros(zb)
    _zero_table(zb, tab_sh, sid, nrb)

    def obody(r, _):
        for c in range(ones_b.shape[1] // 16):
            ones_b[r, pl.ds(c * 16, 16)] = jnp.full((16,), 1.0, F32)
        return 0

    lax.fori_loop(0, ones_b.shape[0], obody, 0)
    plsc.subcore_barrier()

    for j in range(-(-nchunks // 32)):
        chunk = j * 32 + wid

        def _do(chunk=chunk):
            off = chunk * CHUNK
            pltpu.sync_copy(sidx_hbm.at[pl.ds(off, CHUNK)], outv.at[0])
            pltpu.sync_copy(ones_b, tab_sh.at[outv.at[0]], add=True)

        pl.when(chunk < nchunks)(_do)

    plsc.subcore_barrier()
    _writeout_table(zb, tab_sh, out_hbm, cid, sid, nrb)


GRP = 25


def _gine_sc_body(h_hbm, e_hbm, src_hbm, dst_hbm, out_hbm,
                  srcb, dstb, dstv, hrows, erows, agg_sh,
                  gsem0, gsem1, esem0, esem1):
    cid = lax.axis_index("c")
    sid = lax.axis_index("s")
    n = h_hbm.shape[0]
    nrb = n // ROWBLK
    wid = sid * 2 + cid
    per_worker = src_hbm.shape[0] // (32 * CHUNK)

    zb = hrows.at[0]
    _fill_zeros(zb)
    _zero_table(zb, agg_sh, sid, nrb)
    plsc.subcore_barrier()

    def issue(tl, grow, hbuf, ebuf, gsem, esem):
        pltpu.async_copy(h_hbm.at[srcb.at[pl.ds(tl * CHUNK, CHUNK)]],
                         hbuf, gsem)
        pltpu.async_copy(
            e_hbm.at[pl.ds((grow + tl) * CHUNK, CHUNK)], ebuf, esem)

    def drain(hbuf, ebuf, gsem, esem):
        pltpu.make_async_copy(h_hbm.at[pl.ds(0, CHUNK)], hbuf, gsem).wait()
        pltpu.make_async_copy(e_hbm.at[pl.ds(0, CHUNK)], ebuf, esem).wait()

    base_row = wid * per_worker
    for g in range(per_worker // GRP):
        grow = base_row + g * GRP
        pltpu.sync_copy(src_hbm.at[pl.ds(grow * CHUNK, GRP * CHUNK)], srcb)
        pltpu.sync_copy(dst_hbm.at[pl.ds(grow * CHUNK, GRP * CHUNK)], dstb)
        issue(0, grow, hrows.at[0], erows.at[0], gsem0, esem0)

        def tb(t, _, grow=grow):
            b = t % 2

            @pl.when(jnp.logical_and(t + 1 < GRP, b == 0))
            def _():
                issue(t + 1, grow, hrows.at[1], erows.at[1], gsem1, esem1)

            @pl.when(jnp.logical_and(t + 1 < GRP, b == 1))
            def _():
                issue(t + 1, grow, hrows.at[0], erows.at[0], gsem0, esem0)

            @pl.when(b == 0)
            def _():
                drain(hrows.at[0], erows.at[0], gsem0, esem0)

            @pl.when(b == 1)
            def _():
                drain(hrows.at[1], erows.at[1], gsem1, esem1)

            for i in range(CHUNK // 16):
                dstv[0, pl.ds(i * 16, 16)] = dstb[pl.ds(t * CHUNK + i * 16, 16)]

            def rbody(r, _):
                for c in range(8):
                    s_ = pl.ds(c * 16, 16)
                    hrows[b, r, s_] = jnp.maximum(
                        hrows[b, r, s_] + erows[b, r, s_], 0.0)
                return 0

            lax.fori_loop(0, CHUNK, rbody, 0)
            pltpu.sync_copy(hrows.at[b], agg_sh.at[dstv.at[0]], add=True)
            return 0

        lax.fori_loop(0, GRP, tb, 0)

    plsc.subcore_barrier()
    _writeout_table(hrows.at[0], agg_sh, out_hbm, cid, sid, nrb)


def _segsum_stage(vals, gidx, sidx, n):
    nv, hdim = vals.shape
    mesh = plsc.VectorSubcoreMesh(core_axis_name="c", subcore_axis_name="s")
    return pl.kernel(
        _segsum_sc_body,
        out_type=jax.ShapeDtypeStruct((2, n, hdim), F32),
        mesh=mesh,
        scratch_types=[
            pltpu.VMEM((CHUNK,), jnp.int32),
            pltpu.VMEM((1, CHUNK), jnp.int32),
            pltpu.VMEM((CHUNK, hdim), F32),
            pltpu.VMEM((ROWBLK, hdim), F32),
            pltpu.VMEM_SHARED((n, hdim), F32),
            pltpu.SemaphoreType.DMA,
        ],
    )(vals, gidx, sidx)


def _count_stage(sidx, n, hdim):
    mesh = plsc.VectorSubcoreMesh(core_axis_name="c", subcore_axis_name="s")
    return pl.kernel(
        _count_sc_body,
        out_type=jax.ShapeDtypeStruct((2, n, hdim), F32),
        mesh=mesh,
        scratch_types=[
            pltpu.VMEM((1, CHUNK), jnp.int32),
            pltpu.VMEM((CHUNK, hdim), F32),
            pltpu.VMEM((ROWBLK, hdim), F32),
            pltpu.VMEM_SHARED((n, hdim), F32),
            pltpu.SemaphoreType.DMA,
        ],
    )(sidx)


def _gine_stage(h, e_enc, src2, dst2):
    n, hdim = h.shape
    mesh = plsc.VectorSubcoreMesh(core_axis_name="c", subcore_axis_name="s")
    return pl.kernel(
        _gine_sc_body,
        out_type=jax.ShapeDtypeStruct((2, n, hdim), F32),
        mesh=mesh,
        scratch_types=[
            pltpu.VMEM((GRP * CHUNK,), jnp.int32),
            pltpu.VMEM((GRP * CHUNK,), jnp.int32),
            pltpu.VMEM((1, CHUNK), jnp.int32),
            pltpu.VMEM((2, CHUNK, hdim), F32),
            pltpu.VMEM((2, CHUNK, hdim), F32),
            pltpu.VMEM_SHARED((n, hdim), F32),
            pltpu.SemaphoreType.DMA,
            pltpu.SemaphoreType.DMA,
            pltpu.SemaphoreType.DMA,
            pltpu.SemaphoreType.DMA,
        ],
    )(h, e_enc, src2, dst2)


def kernel(x, edge_index, edge_attr, fragments, fragments_edge_index, batch, params):
    p = params
    n, _ = x.shape
    e = edge_attr.shape[0]
    hdim = p['atom_W'].shape[1]
    src, dst = edge_index[0], edge_index[1]
    frow, fcol = fragments_edge_index[0], fragments_edge_index[1]
    l0, l1 = p['layers'][0], p['layers'][1]

    h0, xfrag = pl.pallas_call(
        _atom_enc_body,
        out_shape=[
            jax.ShapeDtypeStruct((n, hdim), F32),
            jax.ShapeDtypeStruct((fragments.shape[0], hdim), F32),
        ],
    )(x, p['atom_W'], p['atom_b'].reshape(1, hdim),
      fragments.reshape(-1, 1), p['frag_emb'])

    be = 8000
    e1, e2 = pl.pallas_call(
        _edge_enc_body,
        grid=(e // be,),
        in_specs=[
            pl.BlockSpec((be, edge_attr.shape[1]), lambda i: (i, 0)),
            pl.BlockSpec((edge_attr.shape[1], hdim), lambda i: (0, 0)),
            pl.BlockSpec((1, hdim), lambda i: (0, 0)),
            pl.BlockSpec((edge_attr.shape[1], hdim), lambda i: (0, 0)),
            pl.BlockSpec((1, hdim), lambda i: (0, 0)),
        ],
        out_specs=[
            pl.BlockSpec((be, hdim), lambda i: (i, 0)),
            pl.BlockSpec((be, hdim), lambda i: (i, 0)),
        ],
        out_shape=[
            jax.ShapeDtypeStruct((e, hdim), F32),
            jax.ShapeDtypeStruct((e, hdim), F32),
        ],
    )(edge_attr, l0['bond_W'], l0['bond_b'].reshape(1, hdim),
      l1['bond_W'], l1['bond_b'].reshape(1, hdim))

    s_parts = _segsum_stage(xfrag, fcol, frow, n)
    c_parts = _count_stage(frow, n, hdim)
    h = pl.pallas_call(
        _fold_body,
        out_shape=jax.ShapeDtypeStruct((n, hdim), F32),
    )(h0, s_parts, c_parts)

    for lp, e_enc in ((l0, e1), (l1, e2)):
        agg = _gine_stage(h, e_enc, src, dst)
        h = pl.pallas_call(
            _layer_body,
            out_shape=jax.ShapeDtypeStruct((n, hdim), F32),
        )(h, agg, lp['eps'].reshape(1, 1),
          lp['nn_W1'], lp['nn_b1'].reshape(1, -1),
          lp['nn_g1'].reshape(1, -1), lp['nn_be1'].reshape(1, -1),
          lp['nn_W2'], lp['nn_b2'].reshape(1, -1),
          lp['bn_g'].reshape(1, -1), lp['bn_b'].reshape(1, -1))

    ao, po = p['atom_out'], p['out']
    out = pl.pallas_call(
        _head_body,
        out_shape=jax.ShapeDtypeStruct((64, po['W2'].shape[1]), F32),
    )(h, batch.reshape(1, n), ao['W1'], ao['b1'].reshape(1, -1),
      ao['W2'], ao['b2'].reshape(1, -1),
      po['W1'], po['b1'].reshape(1, -1),
      po['W2'], po['b2'].reshape(1, -1))
    return out


# speedup vs baseline: 1.1377x; 1.1377x over previous
import jax
import jax.numpy as jnp
from jax import lax
from jax.experimental import pallas as pl
from jax.experimental.pallas import tpu as pltpu
from jax.experimental.pallas import tpu_sc as plsc

F32 = jnp.float32
CHUNK = 80
ROWBLK = 80


def _atom_enc_body(x_ref, w_ref, b_ref, frag_ref, femb_ref, o_ref, xf_ref):
    o_ref[...] = (
        jnp.dot(x_ref[...], w_ref[...], preferred_element_type=F32, precision=lax.Precision.HIGHEST) + b_ref[...]
    )
    nv = femb_ref.shape[0]
    oh = (frag_ref[...] == lax.broadcasted_iota(jnp.int32, (1, nv), 1)).astype(F32)
    xf_ref[...] = jnp.dot(oh, femb_ref[...], preferred_element_type=F32, precision=lax.Precision.HIGHEST)


def _edge_enc_body(ea_ref, w1_ref, b1_ref, w2_ref, b2_ref, e1_ref, e2_ref):
    ea = ea_ref[...]
    k = ea.shape[1]
    e1 = jnp.broadcast_to(b1_ref[...], e1_ref.shape)
    e2 = jnp.broadcast_to(b2_ref[...], e2_ref.shape)
    for i in range(k):
        col = ea[:, i:i + 1]
        e1 = e1 + col * w1_ref[i:i + 1, :]
        e2 = e2 + col * w2_ref[i:i + 1, :]
    e1_ref[...] = e1
    e2_ref[...] = e2


def _fold_body(h0_ref, s_ref, c_ref, o_ref):
    s = s_ref[0] + s_ref[1]
    c = c_ref[0] + c_ref[1]
    o_ref[...] = h0_ref[...] + s / jnp.maximum(c, 1.0)


def _layer_body(h_ref, p_ref, eps_ref, w1_ref, b1_ref, g1_ref, be1_ref,
                w2_ref, b2_ref, g2_ref, be2_ref, o_ref):
    z = h_ref[...] * (1.0 + eps_ref[0, 0]) + p_ref[0] + p_ref[1]
    a = jnp.dot(z, w1_ref[...], preferred_element_type=F32, precision=lax.Precision.HIGHEST) + b1_ref[...]
    m = jnp.mean(a, axis=0, keepdims=True)
    v = jnp.mean((a - m) ** 2, axis=0, keepdims=True)
    a = (a - m) * lax.rsqrt(v + 1e-5) * g1_ref[...] + be1_ref[...]
    a = jnp.maximum(a, 0.0)
    b = jnp.dot(a, w2_ref[...], preferred_element_type=F32, precision=lax.Precision.HIGHEST) + b2_ref[...]
    m2 = jnp.mean(b, axis=0, keepdims=True)
    v2 = jnp.mean((b - m2) ** 2, axis=0, keepdims=True)
    b = (b - m2) * lax.rsqrt(v2 + 1e-5) * g2_ref[...] + be2_ref[...]
    o_ref[...] = jnp.maximum(b, 0.0)


def _head_body(h_ref, batch_ref, aw1_ref, ab1_ref, aw2_ref, ab2_ref,
               ow1_ref, ob1_ref, ow2_ref, ob2_ref, o_ref):
    h = jnp.maximum(
        jnp.dot(h_ref[...], aw1_ref[...], preferred_element_type=F32, precision=lax.Precision.HIGHEST) + ab1_ref[...],
        0.0)
    h = jnp.maximum(
        jnp.dot(h, aw2_ref[...], preferred_element_type=F32, precision=lax.Precision.HIGHEST) + ab2_ref[...], 0.0)
    nb = o_ref.shape[0]
    seg = lax.broadcasted_iota(jnp.int32, (nb, 1), 0)
    oht = (seg == batch_ref[...]).astype(F32)
    gs = jnp.dot(oht, h, preferred_element_type=F32)
    gc = jnp.sum(oht, axis=1, keepdims=True)
    g = gs / jnp.maximum(gc, 1.0)
    g = jnp.maximum(
        jnp.dot(g, ow1_ref[...], preferred_element_type=F32, precision=lax.Precision.HIGHEST) + ob1_ref[...], 0.0)
    o_ref[...] = jnp.dot(g, ow2_ref[...], preferred_element_type=F32, precision=lax.Precision.HIGHEST) + ob2_ref[...]


def _fill_zeros(zb_ref):
    ncol = zb_ref.shape[1]

    def body(r, _):
        for c in range(ncol // 16):
            zb_ref[r, pl.ds(c * 16, 16)] = jnp.zeros((16,), F32)
        return 0

    lax.fori_loop(0, zb_ref.shape[0], body, 0)


def _zero_table(zb_ref, tab_sh, sid, nrb):
    for j in range(-(-nrb // 16)):
        b = j * 16 + sid

        def _zb(b=b):
            pltpu.sync_copy(zb_ref, tab_sh.at[pl.ds(b * ROWBLK, ROWBLK)])

        pl.when(b < nrb)(_zb)


def _writeout_table(zb_ref, tab_sh, out_hbm, cid, sid, nrb):
    for j in range(-(-nrb // 16)):
        b = j * 16 + sid

        def _wb(b=b):
            base = b * ROWBLK
            pltpu.sync_copy(tab_sh.at[pl.ds(base, ROWBLK)], zb_ref)
            pltpu.sync_copy(zb_ref, out_hbm.at[cid, pl.ds(base, ROWBLK)])

        pl.when(b < nrb)(_wb)


def _segsum_sc_body(val_hbm, gidx_hbm, sidx_hbm, out_hbm,
                    inv, outv, rows, zb, tab_sh, sem):
    cid = lax.axis_index("c")
    sid = lax.axis_index("s")
    n = out_hbm.shape[1]
    nrb = n // ROWBLK
    nchunks = gidx_hbm.shape[0] // CHUNK
    wid = sid * 2 + cid

    _fill_zeros(zb)
    _zero_table(zb, tab_sh, sid, nrb)
    plsc.subcore_barrier()

    for j in range(-(-nchunks // 32)):
        chunk = j * 32 + wid

        def _do(chunk=chunk):
            off = chunk * CHUNK
            pltpu.sync_copy(gidx_hbm.at[pl.ds(off, CHUNK)], inv)
            pltpu.async_copy(val_hbm.at[inv], rows, sem).wait()
            pltpu.sync_copy(sidx_hbm.at[pl.ds(off, CHUNK)], outv.at[0])
            pltpu.sync_copy(rows, tab_sh.at[outv.at[0]], add=True)

        pl.when(chunk < nchunks)(_do)

    plsc.subcore_barrier()
    _writeout_table(zb, tab_sh, out_hbm, cid, sid, nrb)


def _count_sc_body(sidx_hbm, out_hbm, outv, ones_b, zb, tab_sh, sem):
    cid = lax.axis_index("c")
    sid = lax.axis_index("s")
    n = out_hbm.shape[1]
    nrb = n // ROWBLK
    nchunks = sidx_hbm.shape[0] // CHUNK
    wid = sid * 2 + cid

    _fill_zeros(zb)
    _zero_table(zb, tab_sh, sid, nrb)

    def obody(r, _):
        for c in range(ones_b.shape[1] // 16):
            ones_b[r, pl.ds(c * 16, 16)] = jnp.full((16,), 1.0, F32)
        return 0

    lax.fori_loop(0, ones_b.shape[0], obody, 0)
    plsc.subcore_barrier()

    for j in range(-(-nchunks // 32)):
        chunk = j * 32 + wid

        def _do(chunk=chunk):
            off = chunk * CHUNK
            pltpu.sync_copy(sidx_hbm.at[pl.ds(off, CHUNK)], outv.at[0])
            pltpu.sync_copy(ones_b, tab_sh.at[outv.at[0]], add=True)

        pl.when(chunk < nchunks)(_do)

    plsc.subcore_barrier()
    _writeout_table(zb, tab_sh, out_hbm, cid, sid, nrb)


GRP = 25


def _gine_sc_body(h_hbm, e_hbm, src_hbm, dst_hbm, out_hbm,
                  srcb, dstb, dstv, hrows, erows, agg_sh,
                  gsem0, gsem1, esem0, esem1):
    cid = lax.axis_index("c")
    sid = lax.axis_index("s")
    n = h_hbm.shape[0]
    nrb = n // ROWBLK
    wid = sid * 2 + cid
    per_worker = src_hbm.shape[0] // (32 * CHUNK)

    zb = hrows.at[0]
    _fill_zeros(zb)
    _zero_table(zb, agg_sh, sid, nrb)
    plsc.subcore_barrier()

    def issue(tl, grow, hbuf, ebuf, gsem, esem):
        pltpu.async_copy(h_hbm.at[srcb.at[pl.ds(tl * CHUNK, CHUNK)]],
                         hbuf, gsem)
        pltpu.async_copy(
            e_hbm.at[pl.ds((grow + tl) * CHUNK, CHUNK)], ebuf, esem)

    def drain(hbuf, ebuf, gsem, esem):
        pltpu.make_async_copy(h_hbm.at[pl.ds(0, CHUNK)], hbuf, gsem).wait()
        pltpu.make_async_copy(e_hbm.at[pl.ds(0, CHUNK)], ebuf, esem).wait()

    def stage_dst(t):
        for i in range(CHUNK // 16):
            dstv[0, pl.ds(i * 16, 16)] = dstb[pl.ds(t * CHUNK + i * 16, 16)]

    def compute(hbuf, ebuf):
        def rbody(r, _):
            for c in range(8):
                s_ = pl.ds(c * 16, 16)
                hbuf[r, s_] = jnp.maximum(hbuf[r, s_] + ebuf[r, s_], 0.0)
            return 0

        lax.fori_loop(0, CHUNK, rbody, 0)

    h0b, e0b = hrows.at[0], erows.at[0]
    h1b, e1b = hrows.at[1], erows.at[1]
    base_row = wid * per_worker
    for g in range(per_worker // GRP):
        grow = base_row + g * GRP
        pltpu.sync_copy(src_hbm.at[pl.ds(grow * CHUNK, GRP * CHUNK)], srcb)
        pltpu.sync_copy(dst_hbm.at[pl.ds(grow * CHUNK, GRP * CHUNK)], dstb)
        issue(0, grow, h0b, e0b, gsem0, esem0)

        def pairbody(k, _, grow=grow):
            t0 = 2 * k
            issue(t0 + 1, grow, h1b, e1b, gsem1, esem1)
            stage_dst(t0)
            drain(h0b, e0b, gsem0, esem0)
            compute(h0b, e0b)
            pltpu.sync_copy(h0b, agg_sh.at[dstv.at[0]], add=True)
            issue(t0 + 2, grow, h0b, e0b, gsem0, esem0)
            stage_dst(t0 + 1)
            drain(h1b, e1b, gsem1, esem1)
            compute(h1b, e1b)
            pltpu.sync_copy(h1b, agg_sh.at[dstv.at[0]], add=True)
            return 0

        lax.fori_loop(0, (GRP - 1) // 2, pairbody, 0)
        stage_dst(GRP - 1)
        drain(h0b, e0b, gsem0, esem0)
        compute(h0b, e0b)
        pltpu.sync_copy(h0b, agg_sh.at[dstv.at[0]], add=True)

    plsc.subcore_barrier()
    _writeout_table(hrows.at[0], agg_sh, out_hbm, cid, sid, nrb)


def _segsum_stage(vals, gidx, sidx, n):
    nv, hdim = vals.shape
    mesh = plsc.VectorSubcoreMesh(core_axis_name="c", subcore_axis_name="s")
    return pl.kernel(
        _segsum_sc_body,
        out_type=jax.ShapeDtypeStruct((2, n, hdim), F32),
        mesh=mesh,
        scratch_types=[
            pltpu.VMEM((CHUNK,), jnp.int32),
            pltpu.VMEM((1, CHUNK), jnp.int32),
            pltpu.VMEM((CHUNK, hdim), F32),
            pltpu.VMEM((ROWBLK, hdim), F32),
            pltpu.VMEM_SHARED((n, hdim), F32),
            pltpu.SemaphoreType.DMA,
        ],
    )(vals, gidx, sidx)


def _count_stage(sidx, n, hdim):
    mesh = plsc.VectorSubcoreMesh(core_axis_name="c", subcore_axis_name="s")
    return pl.kernel(
        _count_sc_body,
        out_type=jax.ShapeDtypeStruct((2, n, hdim), F32),
        mesh=mesh,
        scratch_types=[
            pltpu.VMEM((1, CHUNK), jnp.int32),
            pltpu.VMEM((CHUNK, hdim), F32),
            pltpu.VMEM((ROWBLK, hdim), F32),
            pltpu.VMEM_SHARED((n, hdim), F32),
            pltpu.SemaphoreType.DMA,
        ],
    )(sidx)


def _gine_stage(h, e_enc, src2, dst2):
    n, hdim = h.shape
    mesh = plsc.VectorSubcoreMesh(core_axis_name="c", subcore_axis_name="s")
    return pl.kernel(
        _gine_sc_body,
        out_type=jax.ShapeDtypeStruct((2, n, hdim), F32),
        mesh=mesh,
        scratch_types=[
            pltpu.VMEM((GRP * CHUNK,), jnp.int32),
            pltpu.VMEM((GRP * CHUNK,), jnp.int32),
            pltpu.VMEM((1, CHUNK), jnp.int32),
            pltpu.VMEM((2, CHUNK, hdim), F32),
            pltpu.VMEM((2, CHUNK, hdim), F32),
            pltpu.VMEM_SHARED((n, hdim), F32),
            pltpu.SemaphoreType.DMA,
            pltpu.SemaphoreType.DMA,
            pltpu.SemaphoreType.DMA,
            pltpu.SemaphoreType.DMA,
        ],
    )(h, e_enc, src2, dst2)


def kernel(x, edge_index, edge_attr, fragments, fragments_edge_index, batch, params):
    p = params
    n, _ = x.shape
    e = edge_attr.shape[0]
    hdim = p['atom_W'].shape[1]
    src, dst = edge_index[0], edge_index[1]
    frow, fcol = fragments_edge_index[0], fragments_edge_index[1]
    l0, l1 = p['layers'][0], p['layers'][1]

    h0, xfrag = pl.pallas_call(
        _atom_enc_body,
        out_shape=[
            jax.ShapeDtypeStruct((n, hdim), F32),
            jax.ShapeDtypeStruct((fragments.shape[0], hdim), F32),
        ],
    )(x, p['atom_W'], p['atom_b'].reshape(1, hdim),
      fragments.reshape(-1, 1), p['frag_emb'])

    be = 4000
    e1, e2 = pl.pallas_call(
        _edge_enc_body,
        grid=(e // be,),
        in_specs=[
            pl.BlockSpec((be, edge_attr.shape[1]), lambda i: (i, 0)),
            pl.BlockSpec((edge_attr.shape[1], hdim), lambda i: (0, 0)),
            pl.BlockSpec((1, hdim), lambda i: (0, 0)),
            pl.BlockSpec((edge_attr.shape[1], hdim), lambda i: (0, 0)),
            pl.BlockSpec((1, hdim), lambda i: (0, 0)),
        ],
        out_specs=[
            pl.BlockSpec((be, hdim), lambda i: (i, 0)),
            pl.BlockSpec((be, hdim), lambda i: (i, 0)),
        ],
        out_shape=[
            jax.ShapeDtypeStruct((e, hdim), F32),
            jax.ShapeDtypeStruct((e, hdim), F32),
        ],
    )(edge_attr, l0['bond_W'], l0['bond_b'].reshape(1, hdim),
      l1['bond_W'], l1['bond_b'].reshape(1, hdim))

    s_parts = _segsum_stage(xfrag, fcol, frow, n)
    c_parts = _count_stage(frow, n, hdim)
    h = pl.pallas_call(
        _fold_body,
        out_shape=jax.ShapeDtypeStruct((n, hdim), F32),
    )(h0, s_parts, c_parts)

    for lp, e_enc in ((l0, e1), (l1, e2)):
        agg = _gine_stage(h, e_enc, src, dst)
        h = pl.pallas_call(
            _layer_body,
            out_shape=jax.ShapeDtypeStruct((n, hdim), F32),
        )(h, agg, lp['eps'].reshape(1, 1),
          lp['nn_W1'], lp['nn_b1'].reshape(1, -1),
          lp['nn_g1'].reshape(1, -1), lp['nn_be1'].reshape(1, -1),
          lp['nn_W2'], lp['nn_b2'].reshape(1, -1),
          lp['bn_g'].reshape(1, -1), lp['bn_b'].reshape(1, -1))

    ao, po = p['atom_out'], p['out']
    out = pl.pallas_call(
        _head_body,
        out_shape=jax.ShapeDtypeStruct((64, po['W2'].shape[1]), F32),
    )(h, batch.reshape(1, n), ao['W1'], ao['b1'].reshape(1, -1),
      ao['W2'], ao['b2'].reshape(1, -1),
      po['W1'], po['b1'].reshape(1, -1),
      po['W2'], po['b2'].reshape(1, -1))
    return out


# speedup vs baseline: 1.9676x; 1.7294x over previous
import jax
import jax.numpy as jnp
from jax import lax
from jax.experimental import pallas as pl
from jax.experimental.pallas import tpu as pltpu
from jax.experimental.pallas import tpu_sc as plsc

F32 = jnp.float32
CHUNK = 80
ROWBLK = 80


def _atom_enc_body(x_ref, w_ref, b_ref, frag_ref, femb_ref, o_ref, xf_ref):
    o_ref[...] = (
        jnp.dot(x_ref[...], w_ref[...], preferred_element_type=F32, precision=lax.Precision.HIGHEST) + b_ref[...]
    )
    nv = femb_ref.shape[0]
    oh = (frag_ref[...] == lax.broadcasted_iota(jnp.int32, (1, nv), 1)).astype(F32)
    xf_ref[...] = jnp.dot(oh, femb_ref[...], preferred_element_type=F32, precision=lax.Precision.HIGHEST)


def _dot3(a, b):
    ah = a.astype(jnp.bfloat16)
    al = (a - ah.astype(F32)).astype(jnp.bfloat16)
    bh = b.astype(jnp.bfloat16)
    bl = (b - bh.astype(F32)).astype(jnp.bfloat16)
    return (jnp.dot(ah, bh, preferred_element_type=F32)
            + jnp.dot(ah, bl, preferred_element_type=F32)
            + jnp.dot(al, bh, preferred_element_type=F32))


def _edge_enc_body(ea_ref, w1_ref, b1_ref, w2_ref, b2_ref, e1_ref, e2_ref):
    ea = ea_ref[...]
    e1_ref[...] = _dot3(ea, w1_ref[...]) + b1_ref[...]
    e2_ref[...] = _dot3(ea, w2_ref[...]) + b2_ref[...]


def _fold_body(h0_ref, s_ref, c_ref, o_ref):
    s = s_ref[0] + s_ref[1]
    c = c_ref[0] + c_ref[1]
    o_ref[...] = h0_ref[...] + s / jnp.maximum(c, 1.0)


def _layer_body(h_ref, p_ref, eps_ref, w1_ref, b1_ref, g1_ref, be1_ref,
                w2_ref, b2_ref, g2_ref, be2_ref, o_ref):
    z = h_ref[...] * (1.0 + eps_ref[0, 0]) + p_ref[0] + p_ref[1]
    a = jnp.dot(z, w1_ref[...], preferred_element_type=F32, precision=lax.Precision.HIGHEST) + b1_ref[...]
    m = jnp.mean(a, axis=0, keepdims=True)
    v = jnp.mean((a - m) ** 2, axis=0, keepdims=True)
    a = (a - m) * lax.rsqrt(v + 1e-5) * g1_ref[...] + be1_ref[...]
    a = jnp.maximum(a, 0.0)
    b = jnp.dot(a, w2_ref[...], preferred_element_type=F32, precision=lax.Precision.HIGHEST) + b2_ref[...]
    m2 = jnp.mean(b, axis=0, keepdims=True)
    v2 = jnp.mean((b - m2) ** 2, axis=0, keepdims=True)
    b = (b - m2) * lax.rsqrt(v2 + 1e-5) * g2_ref[...] + be2_ref[...]
    o_ref[...] = jnp.maximum(b, 0.0)


def _head_body(h_ref, batch_ref, aw1_ref, ab1_ref, aw2_ref, ab2_ref,
               ow1_ref, ob1_ref, ow2_ref, ob2_ref, o_ref):
    h = jnp.maximum(
        jnp.dot(h_ref[...], aw1_ref[...], preferred_element_type=F32, precision=lax.Precision.HIGHEST) + ab1_ref[...],
        0.0)
    h = jnp.maximum(
        jnp.dot(h, aw2_ref[...], preferred_element_type=F32, precision=lax.Precision.HIGHEST) + ab2_ref[...], 0.0)
    nb = o_ref.shape[0]
    seg = lax.broadcasted_iota(jnp.int32, (nb, 1), 0)
    oht = (seg == batch_ref[...]).astype(F32)
    gs = jnp.dot(oht, h, preferred_element_type=F32)
    gc = jnp.sum(oht, axis=1, keepdims=True)
    g = gs / jnp.maximum(gc, 1.0)
    g = jnp.maximum(
        jnp.dot(g, ow1_ref[...], preferred_element_type=F32, precision=lax.Precision.HIGHEST) + ob1_ref[...], 0.0)
    o_ref[...] = jnp.dot(g, ow2_ref[...], preferred_element_type=F32, precision=lax.Precision.HIGHEST) + ob2_ref[...]


def _fill_zeros(zb_ref):
    ncol = zb_ref.shape[1]

    def body(r, _):
        for c in range(ncol // 16):
            zb_ref[r, pl.ds(c * 16, 16)] = jnp.zeros((16,), F32)
        return 0

    lax.fori_loop(0, zb_ref.shape[0], body, 0)


def _zero_table(zb_ref, tab_sh, sid, nrb):
    for j in range(-(-nrb // 16)):
        b = j * 16 + sid

        def _zb(b=b):
            pltpu.sync_copy(zb_ref, tab_sh.at[pl.ds(b * ROWBLK, ROWBLK)])

        pl.when(b < nrb)(_zb)


def _writeout_table(zb_ref, tab_sh, out_hbm, cid, sid, nrb):
    for j in range(-(-nrb // 16)):
        b = j * 16 + sid

        def _wb(b=b):
            base = b * ROWBLK
            pltpu.sync_copy(tab_sh.at[pl.ds(base, ROWBLK)], zb_ref)
            pltpu.sync_copy(zb_ref, out_hbm.at[cid, pl.ds(base, ROWBLK)])

        pl.when(b < nrb)(_wb)


def _segsum_sc_body(val_hbm, gidx_hbm, sidx_hbm, out_hbm,
                    inv, outv, rows, zb, tab_sh, sem):
    cid = lax.axis_index("c")
    sid = lax.axis_index("s")
    n = out_hbm.shape[1]
    nrb = n // ROWBLK
    nchunks = gidx_hbm.shape[0] // CHUNK
    wid = sid * 2 + cid

    _fill_zeros(zb)
    _zero_table(zb, tab_sh, sid, nrb)
    plsc.subcore_barrier()

    for j in range(-(-nchunks // 32)):
        chunk = j * 32 + wid

        def _do(chunk=chunk):
            off = chunk * CHUNK
            pltpu.sync_copy(gidx_hbm.at[pl.ds(off, CHUNK)], inv)
            pltpu.async_copy(val_hbm.at[inv], rows, sem).wait()
            pltpu.sync_copy(sidx_hbm.at[pl.ds(off, CHUNK)], outv.at[0])
            pltpu.sync_copy(rows, tab_sh.at[outv.at[0]], add=True)

        pl.when(chunk < nchunks)(_do)

    plsc.subcore_barrier()
    _writeout_table(zb, tab_sh, out_hbm, cid, sid, nrb)


def _count_sc_body(sidx_hbm, out_hbm, outv, ones_b, zb, tab_sh, sem):
    cid = lax.axis_index("c")
    sid = lax.axis_index("s")
    n = out_hbm.shape[1]
    nrb = n // ROWBLK
    nchunks = sidx_hbm.shape[0] // CHUNK
    wid = sid * 2 + cid

    _fill_zeros(zb)
    _zero_table(zb, tab_sh, sid, nrb)

    def obody(r, _):
        for c in range(ones_b.shape[1] // 16):
            ones_b[r, pl.ds(c * 16, 16)] = jnp.full((16,), 1.0, F32)
        return 0

    lax.fori_loop(0, ones_b.shape[0], obody, 0)
    plsc.subcore_barrier()

    for j in range(-(-nchunks // 32)):
        chunk = j * 32 + wid

        def _do(chunk=chunk):
            off = chunk * CHUNK
            pltpu.sync_copy(sidx_hbm.at[pl.ds(off, CHUNK)], outv.at[0])
            pltpu.sync_copy(ones_b, tab_sh.at[outv.at[0]], add=True)

        pl.when(chunk < nchunks)(_do)

    plsc.subcore_barrier()
    _writeout_table(zb, tab_sh, out_hbm, cid, sid, nrb)


GRP = 25


def _gine_sc_body(h_hbm, e_hbm, src_hbm, dst_hbm, out_hbm,
                  srcb, dstb, dstv, hrows, erows, agg_sh,
                  gsem0, gsem1, esem0, esem1):
    cid = lax.axis_index("c")
    sid = lax.axis_index("s")
    n = h_hbm.shape[0]
    nrb = n // ROWBLK
    wid = sid * 2 + cid
    per_worker = src_hbm.shape[0] // (32 * CHUNK)

    zb = hrows.at[0]
    _fill_zeros(zb)
    _zero_table(zb, agg_sh, sid, nrb)
    plsc.subcore_barrier()

    def issue(tl, grow, hbuf, ebuf, gsem, esem):
        pltpu.async_copy(h_hbm.at[srcb.at[pl.ds(tl * CHUNK, CHUNK)]],
                         hbuf, gsem)
        pltpu.async_copy(
            e_hbm.at[pl.ds((grow + tl) * CHUNK, CHUNK)], ebuf, esem)

    def drain(hbuf, ebuf, gsem, esem):
        pltpu.make_async_copy(h_hbm.at[pl.ds(0, CHUNK)], hbuf, gsem).wait()
        pltpu.make_async_copy(e_hbm.at[pl.ds(0, CHUNK)], ebuf, esem).wait()

    def stage_dst(t):
        for i in range(CHUNK // 16):
            dstv[0, pl.ds(i * 16, 16)] = dstb[pl.ds(t * CHUNK + i * 16, 16)]

    def compute(hbuf, ebuf):
        def rbody(r, _):
            for c in range(8):
                s_ = pl.ds(c * 16, 16)
                hbuf[r, s_] = jnp.maximum(hbuf[r, s_] + ebuf[r, s_], 0.0)
            return 0

        lax.fori_loop(0, CHUNK, rbody, 0)

    h0b, e0b = hrows.at[0], erows.at[0]
    h1b, e1b = hrows.at[1], erows.at[1]
    base_row = wid * per_worker
    for g in range(per_worker // GRP):
        grow = base_row + g * GRP
        pltpu.sync_copy(src_hbm.at[pl.ds(grow * CHUNK, GRP * CHUNK)], srcb)
        pltpu.sync_copy(dst_hbm.at[pl.ds(grow * CHUNK, GRP * CHUNK)], dstb)
        issue(0, grow, h0b, e0b, gsem0, esem0)

        def pairbody(k, _, grow=grow):
            t0 = 2 * k
            issue(t0 + 1, grow, h1b, e1b, gsem1, esem1)
            stage_dst(t0)
            drain(h0b, e0b, gsem0, esem0)
            compute(h0b, e0b)
            pltpu.sync_copy(h0b, agg_sh.at[dstv.at[0]], add=True)
            issue(t0 + 2, grow, h0b, e0b, gsem0, esem0)
            stage_dst(t0 + 1)
            drain(h1b, e1b, gsem1, esem1)
            compute(h1b, e1b)
            pltpu.sync_copy(h1b, agg_sh.at[dstv.at[0]], add=True)
            return 0

        lax.fori_loop(0, (GRP - 1) // 2, pairbody, 0)
        stage_dst(GRP - 1)
        drain(h0b, e0b, gsem0, esem0)
        compute(h0b, e0b)
        pltpu.sync_copy(h0b, agg_sh.at[dstv.at[0]], add=True)

    plsc.subcore_barrier()
    _writeout_table(hrows.at[0], agg_sh, out_hbm, cid, sid, nrb)


def _segsum_stage(vals, gidx, sidx, n):
    nv, hdim = vals.shape
    mesh = plsc.VectorSubcoreMesh(core_axis_name="c", subcore_axis_name="s")
    return pl.kernel(
        _segsum_sc_body,
        out_type=jax.ShapeDtypeStruct((2, n, hdim), F32),
        mesh=mesh,
        scratch_types=[
            pltpu.VMEM((CHUNK,), jnp.int32),
            pltpu.VMEM((1, CHUNK), jnp.int32),
            pltpu.VMEM((CHUNK, hdim), F32),
            pltpu.VMEM((ROWBLK, hdim), F32),
            pltpu.VMEM_SHARED((n, hdim), F32),
            pltpu.SemaphoreType.DMA,
        ],
    )(vals, gidx, sidx)


def _count_stage(sidx, n, hdim):
    mesh = plsc.VectorSubcoreMesh(core_axis_name="c", subcore_axis_name="s")
    return pl.kernel(
        _count_sc_body,
        out_type=jax.ShapeDtypeStruct((2, n, hdim), F32),
        mesh=mesh,
        scratch_types=[
            pltpu.VMEM((1, CHUNK), jnp.int32),
            pltpu.VMEM((CHUNK, hdim), F32),
            pltpu.VMEM((ROWBLK, hdim), F32),
            pltpu.VMEM_SHARED((n, hdim), F32),
            pltpu.SemaphoreType.DMA,
        ],
    )(sidx)


def _gine_stage(h, e_enc, src2, dst2):
    n, hdim = h.shape
    mesh = plsc.VectorSubcoreMesh(core_axis_name="c", subcore_axis_name="s")
    return pl.kernel(
        _gine_sc_body,
        out_type=jax.ShapeDtypeStruct((2, n, hdim), F32),
        mesh=mesh,
        scratch_types=[
            pltpu.VMEM((GRP * CHUNK,), jnp.int32),
            pltpu.VMEM((GRP * CHUNK,), jnp.int32),
            pltpu.VMEM((1, CHUNK), jnp.int32),
            pltpu.VMEM((2, CHUNK, hdim), F32),
            pltpu.VMEM((2, CHUNK, hdim), F32),
            pltpu.VMEM_SHARED((n, hdim), F32),
            pltpu.SemaphoreType.DMA,
            pltpu.SemaphoreType.DMA,
            pltpu.SemaphoreType.DMA,
            pltpu.SemaphoreType.DMA,
        ],
    )(h, e_enc, src2, dst2)


def kernel(x, edge_index, edge_attr, fragments, fragments_edge_index, batch, params):
    p = params
    n, _ = x.shape
    e = edge_attr.shape[0]
    hdim = p['atom_W'].shape[1]
    src, dst = edge_index[0], edge_index[1]
    frow, fcol = fragments_edge_index[0], fragments_edge_index[1]
    l0, l1 = p['layers'][0], p['layers'][1]

    h0, xfrag = pl.pallas_call(
        _atom_enc_body,
        out_shape=[
            jax.ShapeDtypeStruct((n, hdim), F32),
            jax.ShapeDtypeStruct((fragments.shape[0], hdim), F32),
        ],
    )(x, p['atom_W'], p['atom_b'].reshape(1, hdim),
      fragments.reshape(-1, 1), p['frag_emb'])

    be = 4000
    e1, e2 = pl.pallas_call(
        _edge_enc_body,
        grid=(e // be,),
        in_specs=[
            pl.BlockSpec((be, edge_attr.shape[1]), lambda i: (i, 0)),
            pl.BlockSpec((edge_attr.shape[1], hdim), lambda i: (0, 0)),
            pl.BlockSpec((1, hdim), lambda i: (0, 0)),
            pl.BlockSpec((edge_attr.shape[1], hdim), lambda i: (0, 0)),
            pl.BlockSpec((1, hdim), lambda i: (0, 0)),
        ],
        out_specs=[
            pl.BlockSpec((be, hdim), lambda i: (i, 0)),
            pl.BlockSpec((be, hdim), lambda i: (i, 0)),
        ],
        out_shape=[
            jax.ShapeDtypeStruct((e, hdim), F32),
            jax.ShapeDtypeStruct((e, hdim), F32),
        ],
    )(edge_attr, l0['bond_W'], l0['bond_b'].reshape(1, hdim),
      l1['bond_W'], l1['bond_b'].reshape(1, hdim))

    s_parts = _segsum_stage(xfrag, fcol, frow, n)
    c_parts = _count_stage(frow, n, hdim)
    h = pl.pallas_call(
        _fold_body,
        out_shape=jax.ShapeDtypeStruct((n, hdim), F32),
    )(h0, s_parts, c_parts)

    for lp, e_enc in ((l0, e1), (l1, e2)):
        agg = _gine_stage(h, e_enc, src, dst)
        h = pl.pallas_call(
            _layer_body,
            out_shape=jax.ShapeDtypeStruct((n, hdim), F32),
        )(h, agg, lp['eps'].reshape(1, 1),
          lp['nn_W1'], lp['nn_b1'].reshape(1, -1),
          lp['nn_g1'].reshape(1, -1), lp['nn_be1'].reshape(1, -1),
          lp['nn_W2'], lp['nn_b2'].reshape(1, -1),
          lp['bn_g'].reshape(1, -1), lp['bn_b'].reshape(1, -1))

    ao, po = p['atom_out'], p['out']
    out = pl.pallas_call(
        _head_body,
        out_shape=jax.ShapeDtypeStruct((64, po['W2'].shape[1]), F32),
    )(h, batch.reshape(1, n), ao['W1'], ao['b1'].reshape(1, -1),
      ao['W2'], ao['b2'].reshape(1, -1),
      po['W1'], po['b1'].reshape(1, -1),
      po['W2'], po['b2'].reshape(1, -1))
    return out
